# baseline (speedup 1.0000x reference)
import functools

import jax
import jax.numpy as jnp
from jax import lax
from jax.experimental import pallas as pl
from jax.experimental.pallas import tpu as pltpu
from jax.experimental.pallas import tpu_sc as plsc

MAX_LEN = 8192
EMBED_DIM = 1024
SEQ_LEN = 4096

_NUM_WORKERS = 32
_ROWS_PER_WORKER = SEQ_LEN // _NUM_WORKERS

_mesh = plsc.VectorSubcoreMesh(core_axis_name="c", subcore_axis_name="s")


@functools.partial(
    pl.kernel,
    mesh=_mesh,
    out_type=jax.ShapeDtypeStruct((SEQ_LEN, EMBED_DIM), jnp.float32),
)
def _copy_rows(table_hbm, out_hbm):
    wid = lax.axis_index("s") * 2 + lax.axis_index("c")
    base = wid * _ROWS_PER_WORKER
    pltpu.sync_copy(
        table_hbm.at[pl.ds(base, _ROWS_PER_WORKER)],
        out_hbm.at[pl.ds(base, _ROWS_PER_WORKER)],
    )


def kernel(seq_len, table):
    del seq_len
    return _copy_rows(table)[None]
